# Initial kernel scaffold; baseline (speedup 1.0000x reference)
import jax
import jax.numpy as jnp
from jax.experimental import pallas as pl


def kernel(x_start, t, noise, sqrt_alphas_cumprod, sqrt_one_minus_alphas_cumprod):
    raise NotImplementedError("write your pallas kernel here")


# speedup vs baseline: 1.9461x; 1.9461x over previous
import functools

import jax
import jax.numpy as jnp
from jax import lax
from jax.experimental import pallas as pl
from jax.experimental.pallas import tpu as pltpu
from jax.experimental.pallas import tpu_sc as plsc

_NC = 2
_NS = 16
_L = 16
_NW = _NC * _NS


def _sc_gather_coeffs(t, table_a, table_b):
    B = t.shape[0]
    per_w = B // _NW
    n_tab = table_a.shape[0]
    mesh = plsc.VectorSubcoreMesh(
        core_axis_name="c", subcore_axis_name="s",
        num_cores=_NC, num_subcores=_NS,
    )

    chunk = 128
    n_chunks = per_w // chunk

    @functools.partial(
        pl.kernel,
        out_type=(
            jax.ShapeDtypeStruct((B,), jnp.float32),
            jax.ShapeDtypeStruct((B,), jnp.float32),
        ),
        mesh=mesh,
        scratch_types=[
            pltpu.VMEM((per_w,), jnp.int32),
            pltpu.VMEM((per_w,), jnp.float32),
            pltpu.VMEM((per_w,), jnp.float32),
            pltpu.SemaphoreType.DMA,
        ],
    )
    def sc_kernel(t_hbm, ta_hbm, tb_hbm, oa_hbm, ob_hbm,
                  idx_v, a_v, b_v, sem):
        wid = lax.axis_index("s") * _NC + lax.axis_index("c")
        base = wid * per_w
        pltpu.sync_copy(t_hbm.at[pl.ds(base, per_w)], idx_v)
        copies = []
        for j in range(n_chunks):
            sl = pl.ds(j * chunk, chunk)
            copies.append(pltpu.async_copy(ta_hbm.at[idx_v.at[sl]], a_v.at[sl], sem))
            copies.append(pltpu.async_copy(tb_hbm.at[idx_v.at[sl]], b_v.at[sl], sem))
        for c in copies:
            c.wait()
        pltpu.sync_copy(a_v, oa_hbm.at[pl.ds(base, per_w)])
        pltpu.sync_copy(b_v, ob_hbm.at[pl.ds(base, per_w)])

    return sc_kernel(t, table_a, table_b)


def _fma_body(a_ref, b_ref, x_ref, n_ref, o_ref):
    o_ref[...] = a_ref[...] * x_ref[...] + b_ref[...] * n_ref[...]


def _tc_fma(a_col, b_col, x_start, noise, blk):
    B, D = x_start.shape
    return pl.pallas_call(
        _fma_body,
        grid=(B // blk,),
        in_specs=[
            pl.BlockSpec((blk, 1), lambda i: (i, 0)),
            pl.BlockSpec((blk, 1), lambda i: (i, 0)),
            pl.BlockSpec((blk, D), lambda i: (i, 0)),
            pl.BlockSpec((blk, D), lambda i: (i, 0)),
        ],
        out_specs=pl.BlockSpec((blk, D), lambda i: (i, 0)),
        out_shape=jax.ShapeDtypeStruct((B, D), jnp.float32),
    )(a_col, b_col, x_start, noise)


def kernel(x_start, t, noise, sqrt_alphas_cumprod, sqrt_one_minus_alphas_cumprod):
    B, _ = x_start.shape
    n_tab = sqrt_alphas_cumprod.shape[0]
    pad = (-n_tab) % 128
    ta = jnp.pad(sqrt_alphas_cumprod, (0, pad))
    tb = jnp.pad(sqrt_one_minus_alphas_cumprod, (0, pad))
    a_c, b_c = _sc_gather_coeffs(t, ta, tb)
    return _tc_fma(a_c.reshape(B, 1), b_c.reshape(B, 1), x_start, noise, blk=512)
